# baseline (speedup 1.0000x reference)
import functools

import jax
import jax.numpy as jnp
from jax import lax
from jax.experimental import pallas as pl
from jax.experimental.pallas import tpu as pltpu
from jax.experimental.pallas import tpu_sc as plsc


@functools.cache
def _make_gather(V: int, D: int, B: int):
    info = plsc.get_sparse_core_info()
    NC, NS = info.num_cores, info.num_subcores
    NW = NC * NS
    assert B % (8 * NW) == 0
    b_per_w = B // NW
    mesh = plsc.VectorSubcoreMesh(core_axis_name="c", subcore_axis_name="s")

    NCH = 1
    assert b_per_w % NCH == 0
    rpc = b_per_w // NCH

    @functools.partial(
        pl.kernel,
        mesh=mesh,
        out_type=jax.ShapeDtypeStruct((B, D), jnp.float32),
        scratch_types=[
            pltpu.VMEM_SHARED((V, D), jnp.float32),
            pltpu.VMEM((b_per_w,), jnp.int32),
            pltpu.VMEM((rpc, D), jnp.float32),
            pltpu.VMEM((rpc if NCH >= 2 else 8, D), jnp.float32),
            pltpu.SemaphoreType.DMA,
            pltpu.SemaphoreType.DMA,
            pltpu.SemaphoreType.DMA,
            pltpu.SemaphoreType.DMA,
        ],
    )
    def gather_kernel(idx_hbm, table_hbm, out_hbm, table_sp, idx_v,
                      buf0, buf1, gsem0, gsem1, wsem0, wsem1):
        cid = lax.axis_index("c")
        sid = lax.axis_index("s")
        wid = sid * NC + cid
        base = wid * b_per_w
        bufs = (buf0, buf1)
        gsems = (gsem0, gsem1)
        wsems = (wsem0, wsem1)

        @pl.when(sid == 0)
        def _stage_table():
            pltpu.sync_copy(table_hbm, table_sp)

        pltpu.sync_copy(idx_hbm.at[pl.ds(base, b_per_w)], idx_v)
        plsc.subcore_barrier()

        g = [None] * NCH
        w = [None] * NCH
        for c in range(NCH):
            b = c % 2
            if c >= 2:
                w[c - 2].wait()
            g[c] = pltpu.async_copy(
                table_sp.at[idx_v.at[pl.ds(c * rpc, rpc)]], bufs[b], gsems[b])
            if c >= 1:
                g[c - 1].wait()
                w[c - 1] = pltpu.async_copy(
                    bufs[1 - b], out_hbm.at[pl.ds(base + (c - 1) * rpc, rpc)],
                    wsems[1 - b])
        g[NCH - 1].wait()
        w[NCH - 1] = pltpu.async_copy(
            bufs[(NCH - 1) % 2],
            out_hbm.at[pl.ds(base + (NCH - 1) * rpc, rpc)],
            wsems[(NCH - 1) % 2])
        if NCH >= 2:
            w[NCH - 2].wait()
        w[NCH - 1].wait()

    return gather_kernel


def kernel(timesteps, embed_table):
    idx = timesteps.astype(jnp.int32)
    fn = _make_gather(embed_table.shape[0], embed_table.shape[1], idx.shape[0])
    return fn(idx, embed_table)


# speedup vs baseline: 1.0246x; 1.0246x over previous
import functools

import jax
import jax.numpy as jnp
from jax import lax
from jax.experimental import pallas as pl
from jax.experimental.pallas import tpu as pltpu
from jax.experimental.pallas import tpu_sc as plsc


@functools.cache
def _make_gather(V: int, D: int, B: int):
    info = plsc.get_sparse_core_info()
    NC, NS = info.num_cores, info.num_subcores
    NW = NC * NS
    assert B % (8 * NW) == 0
    b_per_w = B // NW
    mesh = plsc.VectorSubcoreMesh(core_axis_name="c", subcore_axis_name="s")

    NCH = 4
    ROWS_MAIN = (-(-V // NS) + 7) // 8 * 8
    ROWS_LAST = V - ROWS_MAIN * (NS - 1)
    assert ROWS_LAST > 0
    assert b_per_w % NCH == 0
    rpc = b_per_w // NCH

    @functools.partial(
        pl.kernel,
        mesh=mesh,
        out_type=jax.ShapeDtypeStruct((B, D), jnp.float32),
        scratch_types=[
            pltpu.VMEM_SHARED((V, D), jnp.float32),
            pltpu.VMEM((b_per_w,), jnp.int32),
            pltpu.VMEM((rpc, D), jnp.float32),
            pltpu.VMEM((rpc if NCH >= 2 else 8, D), jnp.float32),
            pltpu.SemaphoreType.DMA,
            pltpu.SemaphoreType.DMA,
            pltpu.SemaphoreType.DMA,
            pltpu.SemaphoreType.DMA,
        ],
    )
    def gather_kernel(idx_hbm, table_hbm, out_hbm, table_sp, idx_v,
                      buf0, buf1, gsem0, gsem1, wsem0, wsem1):
        cid = lax.axis_index("c")
        sid = lax.axis_index("s")
        wid = sid * NC + cid
        base = wid * b_per_w
        bufs = (buf0, buf1)
        gsems = (gsem0, gsem1)
        wsems = (wsem0, wsem1)

        pltpu.sync_copy(idx_hbm.at[pl.ds(base, b_per_w)], idx_v)

        @pl.when(sid < NS - 1)
        def _stage_main():
            r0 = sid * ROWS_MAIN
            pltpu.sync_copy(table_hbm.at[pl.ds(r0, ROWS_MAIN)],
                            table_sp.at[pl.ds(r0, ROWS_MAIN)])

        @pl.when(sid == NS - 1)
        def _stage_last():
            r0 = (NS - 1) * ROWS_MAIN
            pltpu.sync_copy(table_hbm.at[pl.ds(r0, ROWS_LAST)],
                            table_sp.at[pl.ds(r0, ROWS_LAST)])

        plsc.subcore_barrier()

        g = [None] * NCH
        w = [None] * NCH
        for c in range(NCH):
            b = c % 2
            if c >= 2:
                w[c - 2].wait()
            g[c] = pltpu.async_copy(
                table_sp.at[idx_v.at[pl.ds(c * rpc, rpc)]], bufs[b], gsems[b])
            if c >= 1:
                g[c - 1].wait()
                w[c - 1] = pltpu.async_copy(
                    bufs[1 - b], out_hbm.at[pl.ds(base + (c - 1) * rpc, rpc)],
                    wsems[1 - b])
        g[NCH - 1].wait()
        w[NCH - 1] = pltpu.async_copy(
            bufs[(NCH - 1) % 2],
            out_hbm.at[pl.ds(base + (NCH - 1) * rpc, rpc)],
            wsems[(NCH - 1) % 2])
        if NCH >= 2:
            w[NCH - 2].wait()
        w[NCH - 1].wait()

    return gather_kernel


def kernel(timesteps, embed_table):
    idx = timesteps.astype(jnp.int32)
    fn = _make_gather(embed_table.shape[0], embed_table.shape[1], idx.shape[0])
    return fn(idx, embed_table)
